# Initial kernel scaffold; baseline (speedup 1.0000x reference)
import jax
import jax.numpy as jnp
from jax.experimental import pallas as pl


def kernel(node_features, adjacency_matrix, W, b):
    raise NotImplementedError("write your pallas kernel here")


# speedup vs baseline: 17.2055x; 17.2055x over previous
import functools

import jax
import jax.numpy as jnp
from jax import lax
from jax.experimental import pallas as pl
from jax.experimental.pallas import tpu as pltpu
from jax.experimental.pallas import tpu_sc as plsc

N_NODES = 10000
D = 128
NC = 2
NS = 16
NW = NC * NS
CHUNK = 80
N_ACC = 10240
PAD_DST = N_NODES + 8

_mesh = plsc.VectorSubcoreMesh(core_axis_name="c", subcore_axis_name="s")


def _make_deg_kernel(n_edges):
    epw = n_edges // NW
    n_chunks = epw // CHUNK

    @functools.partial(
        pl.kernel,
        out_type=[jax.ShapeDtypeStruct((N_ACC,), jnp.float32),
                  jax.ShapeDtypeStruct((N_ACC,), jnp.float32)],
        mesh=_mesh,
        scratch_types=[
            pltpu.VMEM((CHUNK,), jnp.int32),
            pltpu.VMEM((CHUNK,), jnp.float32),
            pltpu.VMEM((N_ACC // NS,), jnp.float32),
            pltpu.VMEM_SHARED((N_ACC,), jnp.float32),
        ],
    )
    def deg_kernel(dst_hbm, out0_hbm, out1_hbm, idx_v, ones_v, zbuf, sdeg):
        c = lax.axis_index("c")
        s = lax.axis_index("s")
        slc = N_ACC // NS

        def _zero(i, _):
            zbuf[pl.ds(i * 16, 16)] = jnp.zeros((16,), jnp.float32)
            return 0
        lax.fori_loop(0, slc // 16, _zero, 0)
        for j in range(CHUNK // 16):
            ones_v[pl.ds(j * 16, 16)] = jnp.ones((16,), jnp.float32)
        pltpu.sync_copy(zbuf, sdeg.at[pl.ds(s * slc, slc)])
        plsc.subcore_barrier()

        base = (c * NS + s) * epw

        def _body(i, _):
            pltpu.sync_copy(dst_hbm.at[pl.ds(base + i * CHUNK, CHUNK)], idx_v)
            pltpu.sync_copy(ones_v, sdeg.at[idx_v], add=True)
            return 0
        lax.fori_loop(0, n_chunks, _body, 0)
        plsc.subcore_barrier()

        pltpu.sync_copy(sdeg.at[pl.ds(s * slc, slc)], zbuf)

        @pl.when(c == 0)
        def _():
            pltpu.sync_copy(zbuf, out0_hbm.at[pl.ds(s * slc, slc)])

        @pl.when(c == 1)
        def _():
            pltpu.sync_copy(zbuf, out1_hbm.at[pl.ds(s * slc, slc)])

    return deg_kernel


def _make_scatter_kernel(n_edges):
    epw = n_edges // NW
    n_chunks = epw // CHUNK
    rows_per_tile = N_ACC // NS
    ZB = 128

    @functools.partial(
        pl.kernel,
        out_type=[jax.ShapeDtypeStruct((N_ACC, D), jnp.float32),
                  jax.ShapeDtypeStruct((N_ACC, D), jnp.float32)],
        mesh=_mesh,
        scratch_types=[
            pltpu.VMEM((CHUNK,), jnp.int32),
            pltpu.VMEM((CHUNK,), jnp.int32),
            pltpu.VMEM((CHUNK, D), jnp.float32),
            pltpu.VMEM((ZB, D), jnp.float32),
            pltpu.VMEM_SHARED((N_ACC, D), jnp.float32),
        ],
    )
    def scatter_kernel(g_hbm, src_hbm, dst_hbm, out0_hbm, out1_hbm,
                       sidx, didx, rows, zbuf, acc):
        c = lax.axis_index("c")
        s = lax.axis_index("s")

        def _zero(i, _):
            zbuf[i // 8, pl.ds((i % 8) * 16, 16)] = jnp.zeros((16,), jnp.float32)
            return 0
        lax.fori_loop(0, ZB * (D // 16), _zero, 0)
        for j in range(rows_per_tile // ZB):
            pltpu.sync_copy(
                zbuf, acc.at[pl.ds(s * rows_per_tile + j * ZB, ZB)])
        plsc.subcore_barrier()

        base = (c * NS + s) * epw

        def _body(i, _):
            off = base + i * CHUNK
            pltpu.sync_copy(src_hbm.at[pl.ds(off, CHUNK)], sidx)
            pltpu.sync_copy(dst_hbm.at[pl.ds(off, CHUNK)], didx)
            pltpu.sync_copy(g_hbm.at[sidx], rows)
            pltpu.sync_copy(rows, acc.at[didx], add=True)
            return 0
        lax.fori_loop(0, n_chunks, _body, 0)
        plsc.subcore_barrier()

        for j in range(rows_per_tile // ZB):
            r0 = s * rows_per_tile + j * ZB
            pltpu.sync_copy(acc.at[pl.ds(r0, ZB)], zbuf)

            @pl.when(c == 0)
            def _():
                pltpu.sync_copy(zbuf, out0_hbm.at[pl.ds(r0, ZB)])

            @pl.when(c == 1)
            def _():
                pltpu.sync_copy(zbuf, out1_hbm.at[pl.ds(r0, ZB)])

    return scatter_kernel


BLK = 400


def _matmul_body(x_ref, w_ref, degp_ref, g_ref):
    deg = degp_ref[:, 0] + degp_ref[:, 1] + 1.0
    dis = lax.rsqrt(deg)
    h = jnp.dot(x_ref[...], w_ref[...], preferred_element_type=jnp.float32)
    g_ref[...] = h * dis[:, None]


def _matmul(x, w, degp_t):
    return pl.pallas_call(
        _matmul_body,
        grid=(N_NODES // BLK,),
        in_specs=[
            pl.BlockSpec((BLK, D), lambda i: (i, 0)),
            pl.BlockSpec((D, D), lambda i: (0, 0)),
            pl.BlockSpec((BLK, NC), lambda i: (i, 0)),
        ],
        out_specs=pl.BlockSpec((BLK, D), lambda i: (i, 0)),
        out_shape=jax.ShapeDtypeStruct((N_NODES, D), jnp.float32),
    )(x, w, degp_t)


def _final_body(p0_ref, p1_ref, g_ref, degp_ref, b_ref, o_ref):
    deg = degp_ref[:, 0] + degp_ref[:, 1] + 1.0
    dis = lax.rsqrt(deg)
    o_ref[...] = (dis[:, None] * (p0_ref[...] + p1_ref[...] + g_ref[...])
                  + b_ref[...])


def _final(p0, p1, g, degp_t, b2d):
    return pl.pallas_call(
        _final_body,
        grid=(N_NODES // BLK,),
        in_specs=[
            pl.BlockSpec((BLK, D), lambda i: (i, 0)),
            pl.BlockSpec((BLK, D), lambda i: (i, 0)),
            pl.BlockSpec((BLK, D), lambda i: (i, 0)),
            pl.BlockSpec((BLK, NC), lambda i: (i, 0)),
            pl.BlockSpec((1, D), lambda i: (0, 0)),
        ],
        out_specs=pl.BlockSpec((BLK, D), lambda i: (i, 0)),
        out_shape=jax.ShapeDtypeStruct((N_NODES, D), jnp.float32),
    )(p0, p1, g, degp_t, b2d)


def kernel(node_features, adjacency_matrix, W, b):
    src = adjacency_matrix[0].astype(jnp.int32)
    dst = adjacency_matrix[1].astype(jnp.int32)
    n_edges = src.shape[0]
    quantum = NW * CHUNK
    n_pad = (-n_edges) % quantum
    if n_pad:
        src = jnp.concatenate([src, jnp.zeros((n_pad,), jnp.int32)])
        dst = jnp.concatenate([dst, jnp.full((n_pad,), PAD_DST, jnp.int32)])
    e_total = n_edges + n_pad

    d0, d1 = _make_deg_kernel(e_total)(dst)
    degp_t = jnp.stack([d0[:N_NODES], d1[:N_NODES]], axis=1)
    g = _matmul(node_features, W, degp_t)
    p0, p1 = _make_scatter_kernel(e_total)(g, src, dst)
    return _final(p0, p1, g, degp_t, b.reshape(1, D))
